# baseline (speedup 1.0000x reference)
import functools

import jax
import jax.numpy as jnp
from jax import lax
from jax.experimental import pallas as pl
from jax.experimental.pallas import tpu as pltpu
from jax.experimental.pallas import tpu_sc as plsc

N = 10000
E = 320000
ND = 128
ED = 16
H = 128
G = 16

NODE_BLK = 2000
EDGE_BLK = 4096
EP = 331776
N_SLICE = 1
EPS = EP // N_SLICE

SC_NC = 2
SC_NS = 16
SC_NW = SC_NC * SC_NS
GC = 128
EPW = E // SC_NW
G_FULL = EPW // GC
G_TAIL = EPW - G_FULL * GC
SC_ROWS = 624
SC_REM = N - SC_ROWS * SC_NS
S_CHUNKS = (E // 2) // GC
S_KMAX = -(-S_CHUNKS // SC_NS)


def _ln_tc(h, g, b):
    m = jnp.mean(h, axis=-1, keepdims=True)
    v = jnp.mean((h - m) * (h - m), axis=-1, keepdims=True)
    return (h - m) * jax.lax.rsqrt(v + 1e-5) * g + b


def _node_kernel(x_ref, ne_ref, w1n_ref, b1n_ref, gn_ref, bn_ref, w2n_ref,
                 b2n_ref, wm_ref, wgxc_ref, wgnc_ref, wgxr_ref, wgnr_ref,
                 tc_ref, tr_ref):
    x = x_ref[...]
    ne = ne_ref[...]
    h = jnp.dot(x, w1n_ref[...], preferred_element_type=jnp.float32) + b1n_ref[...]
    h = _ln_tc(h, gn_ref[...], bn_ref[...])
    h = jnp.maximum(h, 0.0)
    h_node = jnp.dot(h, w2n_ref[...], preferred_element_type=jnp.float32) + b2n_ref[...]
    hnm = jnp.dot(h_node, wm_ref[...], preferred_element_type=jnp.float32)
    a = (jnp.dot(x, wgxc_ref[...], preferred_element_type=jnp.float32)
         + jnp.dot(ne, wgnc_ref[...], preferred_element_type=jnp.float32))
    b = (jnp.dot(x, wgxr_ref[...], preferred_element_type=jnp.float32)
         + jnp.dot(ne, wgnr_ref[...], preferred_element_type=jnp.float32))
    hi = lax.bitcast_convert_type(hnm, jnp.int32)
    ph = (hi + 0x8000) & jnp.int32(-65536)
    ai = lax.bitcast_convert_type(a, jnp.int32)
    bi = lax.bitcast_convert_type(b, jnp.int32)
    pa = lax.shift_right_logical(ai + 0x8000, 16)
    pb = lax.shift_right_logical(bi + 0x8000, 16)
    tc_ref[...] = ph | pa
    tr_ref[...] = ph | pb


def _node_tables(x, ne, w1n, b1n, gn, bn, w2n, b2n, wm, wgxc, wgnc, wgxr, wgnr):
    nblk = N // NODE_BLK
    row_spec = pl.BlockSpec((NODE_BLK, None), lambda i: (i, 0))
    full = lambda s: pl.BlockSpec(s, lambda i: tuple(0 for _ in s))
    return pl.pallas_call(
        _node_kernel,
        grid=(nblk,),
        in_specs=[
            pl.BlockSpec((NODE_BLK, ND), lambda i: (i, 0)),
            pl.BlockSpec((NODE_BLK, G), lambda i: (i, 0)),
            full((ND, H)), full((1, H)), full((1, H)), full((1, H)),
            full((H, H)), full((1, H)), full((H, H)),
            full((ND, H)), full((G, H)), full((ND, H)), full((G, H)),
        ],
        out_specs=[
            pl.BlockSpec((NODE_BLK, H), lambda i: (i, 0)),
            pl.BlockSpec((NODE_BLK, H), lambda i: (i, 0)),
        ],
        out_shape=[
            jax.ShapeDtypeStruct((N, H), jnp.int32),
            jax.ShapeDtypeStruct((N, H), jnp.int32),
        ],
    )(x, ne, w1n, b1n, gn, bn, w2n, b2n, wm, wgxc, wgnc, wgxr, wgnr)


def _edge_kernel(ea_ref, gc_ref, gr_ref, w1x2_ref, b1e_ref, ge_ref, be_ref,
                 w2ep_ref, b2ep_ref, b1g_ref, gg_ref, bg_ref,
                 w2g_ref, b2g_ref, msg_ref):
    ea = ea_ref[...]
    wa = gc_ref[...]
    wb = gr_ref[...]
    g1 = (lax.bitcast_convert_type(lax.shift_left(wa, 16), jnp.float32)
          + lax.bitcast_convert_type(lax.shift_left(wb, 16), jnp.float32))
    g2 = (lax.bitcast_convert_type(wa & jnp.int32(-65536), jnp.float32)
          + lax.bitcast_convert_type(wb & jnp.int32(-65536), jnp.float32))
    both = jnp.dot(ea, w1x2_ref[...], preferred_element_type=jnp.float32)
    eh = both[:, :H] + b1e_ref[...]
    gh = both[:, H:] + b1g_ref[...] + g1
    gh = _ln_tc(gh, gg_ref[...], bg_ref[...])
    gh = jnp.maximum(gh, 0.0).astype(jnp.bfloat16)
    gate = jnp.dot(gh, w2g_ref[...], preferred_element_type=jnp.float32) + b2g_ref[...]
    eh = _ln_tc(eh, ge_ref[...], be_ref[...])
    eh = jnp.maximum(eh, 0.0).astype(jnp.bfloat16)
    me = jnp.dot(eh, w2ep_ref[...], preferred_element_type=jnp.float32) + b2ep_ref[...]
    msg_ref[...] = (me + g2) * jax.nn.sigmoid(gate)


def _edge_msgs(ea, g_c, g_r, w1x2, b1e, ge, be, w2ep, b2ep, b1g, gg, bg,
               w2g, b2g):
    ne = ea.shape[0]
    assert ne % EDGE_BLK == 0
    nblk = ne // EDGE_BLK
    full = lambda s: pl.BlockSpec(s, lambda i: tuple(0 for _ in s))
    return pl.pallas_call(
        _edge_kernel,
        grid=(nblk,),
        in_specs=[
            pl.BlockSpec((EDGE_BLK, ED), lambda i: (i, 0)),
            pl.BlockSpec((EDGE_BLK, H), lambda i: (i, 0)),
            pl.BlockSpec((EDGE_BLK, H), lambda i: (i, 0)),
            full((ED, 2 * H)), full((1, H)), full((1, H)), full((1, H)),
            full((H, H)), full((1, H)),
            full((1, H)), full((1, H)), full((1, H)),
            full((H, H)), full((1, H)),
        ],
        out_specs=pl.BlockSpec((EDGE_BLK, H), lambda i: (i, 0)),
        out_shape=jax.ShapeDtypeStruct((ne, H), jnp.float32),
    )(ea, g_c, g_r, w1x2, b1e, ge, be, w2ep, b2ep, b1g, gg, bg, w2g, b2g)


def _sc_gather(tc, tr, col, row, dep):
    ne = col.shape[0]
    epw = ne // SC_NW
    g_full = epw // GC
    assert ne % SC_NW == 0 and g_full % 3 == 0 and g_full * GC == epw

    def body(dep_hbm, tc_hbm, tr_hbm, col_hbm, row_hbm, gc_hbm, gr_hbm,
             icol, irow, buf_a0, buf_b0, buf_a1, buf_b1, buf_a2, buf_b2,
             sem_g0, sem_g1, sem_g2, sem_w0, sem_w1, sem_w2):
        del dep_hbm
        wid = lax.axis_index("s") * SC_NC + lax.axis_index("c")
        e0 = wid * epw
        pltpu.sync_copy(col_hbm.at[pl.ds(e0, epw)], icol)
        pltpu.sync_copy(row_hbm.at[pl.ds(e0, epw)], irow)

        slots = ((buf_a0, buf_b0, sem_g0, sem_w0),
                 (buf_a1, buf_b1, sem_g1, sem_w1),
                 (buf_a2, buf_b2, sem_g2, sem_w2))

        def issue(k, slot):
            ba, bb, sg, _ = slots[slot]
            pltpu.async_copy(tc_hbm.at[icol.at[pl.ds(k * GC, GC)]], ba, sg)
            pltpu.async_copy(tr_hbm.at[irow.at[pl.ds(k * GC, GC)]], bb, sg)

        issue(0, 0)
        issue(1, 1)
        issue(2, 2)

        def round_(t, carry):
            for b in (0, 1, 2):
                k = 3 * t + b
                ba, bb, sg, sw = slots[b]
                pltpu.make_async_copy(tc_hbm.at[icol.at[pl.ds(0, GC)]], ba, sg).wait()
                pltpu.make_async_copy(tr_hbm.at[irow.at[pl.ds(0, GC)]], bb, sg).wait()
                pltpu.async_copy(ba, gc_hbm.at[pl.ds(e0 + k * GC, GC)], sw)
                pltpu.async_copy(bb, gr_hbm.at[pl.ds(e0 + k * GC, GC)], sw)

                @pl.when(k < g_full - 3)
                def _():
                    pltpu.make_async_copy(ba, gc_hbm.at[pl.ds(0, GC)], sw).wait()
                    pltpu.make_async_copy(bb, gr_hbm.at[pl.ds(0, GC)], sw).wait()
                    issue(k + 3, b)

            return carry

        lax.fori_loop(0, g_full // 3, round_, 0)

        for b in (0, 1, 2):
            ba, bb, sg, sw = slots[b]
            pltpu.make_async_copy(ba, gc_hbm.at[pl.ds(0, GC)], sw).wait()
            pltpu.make_async_copy(bb, gr_hbm.at[pl.ds(0, GC)], sw).wait()

    mesh = plsc.VectorSubcoreMesh(core_axis_name="c", subcore_axis_name="s",
                                  num_cores=SC_NC, num_subcores=SC_NS)
    return pl.kernel(
        body,
        out_type=[jax.ShapeDtypeStruct((ne, H), jnp.int32),
                  jax.ShapeDtypeStruct((ne, H), jnp.int32)],
        mesh=mesh,
        scratch_types=[
            pltpu.VMEM((epw,), jnp.int32),
            pltpu.VMEM((epw,), jnp.int32),
            pltpu.VMEM((GC, H), jnp.int32),
            pltpu.VMEM((GC, H), jnp.int32),
            pltpu.VMEM((GC, H), jnp.int32),
            pltpu.VMEM((GC, H), jnp.int32),
            pltpu.VMEM((GC, H), jnp.int32),
            pltpu.VMEM((GC, H), jnp.int32),
            pltpu.SemaphoreType.DMA,
            pltpu.SemaphoreType.DMA,
            pltpu.SemaphoreType.DMA,
            pltpu.SemaphoreType.DMA,
            pltpu.SemaphoreType.DMA,
            pltpu.SemaphoreType.DMA,
        ],
    )(dep, tc, tr, col, row)


def _make_sc_scatter_body(ne):
  def _sc_scatter_body(dep_hbm, msg_hbm, row_hbm, out_hbm, idx0, idx1, mbuf0,
                       mbuf1, accum, sem_m0, sem_m1):
    del dep_hbm
    core = lax.axis_index("c")
    sid = lax.axis_index("s")

    def zero_row(r, c2):
        for j in range(H // 16):
            mbuf0[r, pl.ds(j * 16, 16)] = jnp.zeros((16,), jnp.float32)
        return c2

    lax.fori_loop(0, GC, zero_row, 0)
    for i in range(SC_ROWS // GC):
        pltpu.sync_copy(mbuf0, accum.at[pl.ds(sid * SC_ROWS + i * GC, GC)])
    rem = SC_ROWS - (SC_ROWS // GC) * GC
    if rem:
        pltpu.sync_copy(mbuf0.at[pl.ds(0, rem)],
                        accum.at[pl.ds(sid * SC_ROWS + (SC_ROWS // GC) * GC,
                                       rem)])

    @pl.when(sid == 0)
    def _():
        pltpu.sync_copy(mbuf0.at[pl.ds(0, SC_REM)],
                        accum.at[pl.ds(SC_ROWS * SC_NS, SC_REM)])

    plsc.subcore_barrier()

    base0 = core * (ne // 2)
    s_chunks = (ne // 2) // GC
    s_kmax = -(-s_chunks // SC_NS)
    slots = ((mbuf0, idx0, sem_m0), (mbuf1, idx1, sem_m1))

    def issue(k, slot):
        mb, ix, sm = slots[slot]
        cid = k * SC_NS + sid
        pltpu.async_copy(msg_hbm.at[pl.ds(base0 + cid * GC, GC)], mb, sm)
        pltpu.async_copy(row_hbm.at[pl.ds(base0 + cid * GC, GC)], ix, sm)

    issue(0, 0)
    issue(1, 1)

    def round_(t, carry):
        for b in (0, 1):
            k = 2 * t + b
            cid = k * SC_NS + sid
            mb, ix, sm = slots[b]

            @pl.when(cid < s_chunks)
            def _():
                pltpu.make_async_copy(msg_hbm.at[pl.ds(0, GC)], mb, sm).wait()
                pltpu.make_async_copy(row_hbm.at[pl.ds(0, GC)], ix, sm).wait()
                pltpu.sync_copy(mb, accum.at[ix], add=True)

            @pl.when((k + 2) * SC_NS + sid < s_chunks)
            def _():
                issue(k + 2, b)

        return carry

    lax.fori_loop(0, (s_kmax + 1) // 2, round_, 0)
    plsc.subcore_barrier()
    pltpu.sync_copy(accum.at[pl.ds(sid * SC_ROWS, SC_ROWS)],
                    out_hbm.at[core, pl.ds(sid * SC_ROWS, SC_ROWS)])

    @pl.when(sid == 0)
    def _():
        pltpu.sync_copy(accum.at[pl.ds(SC_ROWS * SC_NS, SC_REM)],
                        out_hbm.at[core, pl.ds(SC_ROWS * SC_NS, SC_REM)])

  return _sc_scatter_body


def _sc_scatter(msg, row, dep):
    ne = msg.shape[0]
    assert ne % (2 * GC) == 0
    mesh = plsc.VectorSubcoreMesh(core_axis_name="c", subcore_axis_name="s",
                                  num_cores=SC_NC, num_subcores=SC_NS)
    return pl.kernel(
        _make_sc_scatter_body(ne),
        out_type=jax.ShapeDtypeStruct((SC_NC, N, H), jnp.float32),
        mesh=mesh,
        scratch_types=[
            pltpu.VMEM((GC,), jnp.int32),
            pltpu.VMEM((GC,), jnp.int32),
            pltpu.VMEM((GC, H), jnp.float32),
            pltpu.VMEM((GC, H), jnp.float32),
            pltpu.VMEM_SHARED((N + 16, H), jnp.float32),
            pltpu.SemaphoreType.DMA,
            pltpu.SemaphoreType.DMA,
        ],
    )(dep, msg, row)


def _out_kernel(x_ref, *refs):
    (*a_refs, wc_ref, bc_ref, w1o_ref, b1o_ref, go_ref, bo_ref, w2o_ref,
     b2o_ref, lng_ref, lnb_ref, out_ref) = refs
    x = x_ref[...]
    aggr = a_refs[0][...]
    for ar in a_refs[1:]:
        aggr = aggr + ar[...]
    h = (jnp.dot(x, wc_ref[...], preferred_element_type=jnp.float32)
         + bc_ref[...] + aggr)
    h = jnp.dot(h, w1o_ref[...], preferred_element_type=jnp.float32) + b1o_ref[...]
    h = _ln_tc(h, go_ref[...], bo_ref[...])
    h = jnp.maximum(h, 0.0)
    h = jnp.dot(h, w2o_ref[...], preferred_element_type=jnp.float32) + b2o_ref[...]
    out_ref[...] = _ln_tc(h + x, lng_ref[...], lnb_ref[...])


def _out_mlp(x, aggrs, wc, bc, w1o, b1o, go, bo, w2o, b2o, lng, lnb):
    nblk = N // NODE_BLK
    full = lambda s: pl.BlockSpec(s, lambda i: tuple(0 for _ in s))
    return pl.pallas_call(
        _out_kernel,
        grid=(nblk,),
        in_specs=[
            pl.BlockSpec((NODE_BLK, ND), lambda i: (i, 0)),
        ] + [
            pl.BlockSpec((NODE_BLK, H), lambda i: (i, 0))
            for _ in aggrs
        ] + [
            full((ND, H)), full((1, H)),
            full((H, H)), full((1, H)), full((1, H)), full((1, H)),
            full((H, ND)), full((1, ND)), full((1, ND)), full((1, ND)),
        ],
        out_specs=pl.BlockSpec((NODE_BLK, ND), lambda i: (i, 0)),
        out_shape=jax.ShapeDtypeStruct((N, ND), jnp.float32),
    )(x, *aggrs, wc, bc, w1o, b1o, go, bo, w2o, b2o, lng, lnb)


def kernel(x, edge_index, edge_attr, node_extra, params):
    row = edge_index[0].astype(jnp.int32)
    col = edge_index[1].astype(jnp.int32)

    w1n, b1n, gn, bn, w2n, b2n = params['node_net']
    w1e, b1e, ge, be, w2e, b2e = params['edge_net']
    w1g, b1g, gg, bg, w2g, b2g = params['gate']
    wm, bm = params['msg']
    wc, bc = params['cent']
    lng, lnb = params['ln']

    r2 = lambda v: v.reshape(1, -1)
    wgea = w1g[0:ED]
    wgxc = w1g[ED:ED + ND]
    wgnc = w1g[ED + ND:ED + ND + G]
    wgxr = w1g[ED + ND + G:ED + 2 * ND + G]
    wgnr = w1g[ED + 2 * ND + G:]
    w2ep = w2e @ wm
    b2ep = b2e @ wm + bm

    tc, tr = _node_tables(x, node_extra, w1n, r2(b1n), r2(gn), r2(bn), w2n,
                          r2(b2n), wm, wgxc, wgnc, wgxr, wgnr)

    bf = lambda v: v.astype(jnp.bfloat16)
    w1x2 = jnp.concatenate([w1e, wgea], axis=1)
    pad = EP - E
    ea_p = jnp.concatenate([bf(edge_attr),
                            jnp.zeros((pad, ED), jnp.bfloat16)])
    spread = jnp.arange(pad, dtype=jnp.int32) % N
    col_p = jnp.concatenate([col, spread])
    row_g = jnp.concatenate([row, spread])
    row_s = jnp.concatenate([row, jnp.full((pad,), N, jnp.int32)])

    partials = []
    sc_dep = x[:8, :8]
    msgs = []
    for s in range(N_SLICE):
        sl = slice(s * EPS, (s + 1) * EPS)
        g_c, g_r = _sc_gather(tc, tr, col_p[sl], row_g[sl], sc_dep)
        sc_dep = g_c
        msg = _edge_msgs(ea_p[sl], g_c, g_r, bf(w1x2), r2(b1e), r2(ge),
                         r2(be), bf(w2ep), r2(b2ep), r2(b1g), r2(gg), r2(bg),
                         bf(w2g), r2(b2g))
        msgs.append((msg, row_s[sl]))
    for msg, rw in msgs:
        p = _sc_scatter(msg, rw, sc_dep)
        sc_dep = p
        partials.extend([p[0], p[1]])

    w1o, b1o, go, bo, w2o, b2o = params['out_layer']
    return _out_mlp(x, partials, wc, r2(bc), w1o, r2(b1o),
                    r2(go), r2(bo), w2o, r2(b2o), r2(lng), r2(lnb))


# speedup vs baseline: 1.0714x; 1.0714x over previous
import functools

import jax
import jax.numpy as jnp
from jax import lax
from jax.experimental import pallas as pl
from jax.experimental.pallas import tpu as pltpu
from jax.experimental.pallas import tpu_sc as plsc

N = 10000
E = 320000
ND = 128
ED = 16
H = 128
G = 16

NODE_BLK = 2000
EDGE_BLK = 4000
EP = E
N_SLICE = 1
EPS = EP // N_SLICE

SC_NC = 2
SC_NS = 16
SC_NW = SC_NC * SC_NS
GC = 128
EPW = E // SC_NW
G_FULL = EPW // GC
G_TAIL = EPW - G_FULL * GC
SC_ROWS = 624
SC_REM = N - SC_ROWS * SC_NS
S_CHUNKS = (E // 2) // GC
S_KMAX = -(-S_CHUNKS // SC_NS)


def _ln_tc(h, g, b):
    m = jnp.mean(h, axis=-1, keepdims=True)
    v = jnp.mean((h - m) * (h - m), axis=-1, keepdims=True)
    return (h - m) * jax.lax.rsqrt(v + 1e-5) * g + b


def _node_kernel(x_ref, ne_ref, w1n_ref, b1n_ref, gn_ref, bn_ref, w2n_ref,
                 b2n_ref, wm_ref, wgxc_ref, wgnc_ref, wgxr_ref, wgnr_ref,
                 tc_ref, tr_ref):
    x = x_ref[...]
    ne = ne_ref[...]
    h = jnp.dot(x, w1n_ref[...], preferred_element_type=jnp.float32) + b1n_ref[...]
    h = _ln_tc(h, gn_ref[...], bn_ref[...])
    h = jnp.maximum(h, 0.0)
    h_node = jnp.dot(h, w2n_ref[...], preferred_element_type=jnp.float32) + b2n_ref[...]
    hnm = jnp.dot(h_node, wm_ref[...], preferred_element_type=jnp.float32)
    a = (jnp.dot(x, wgxc_ref[...], preferred_element_type=jnp.float32)
         + jnp.dot(ne, wgnc_ref[...], preferred_element_type=jnp.float32))
    b = (jnp.dot(x, wgxr_ref[...], preferred_element_type=jnp.float32)
         + jnp.dot(ne, wgnr_ref[...], preferred_element_type=jnp.float32))
    hi = lax.bitcast_convert_type(hnm, jnp.int32)
    ph = (hi + 0x8000) & jnp.int32(-65536)
    ai = lax.bitcast_convert_type(a, jnp.int32)
    bi = lax.bitcast_convert_type(b, jnp.int32)
    pa = lax.shift_right_logical(ai + 0x8000, 16)
    pb = lax.shift_right_logical(bi + 0x8000, 16)
    tc_ref[...] = ph | pa
    tr_ref[...] = ph | pb


def _node_tables(x, ne, w1n, b1n, gn, bn, w2n, b2n, wm, wgxc, wgnc, wgxr, wgnr):
    nblk = N // NODE_BLK
    row_spec = pl.BlockSpec((NODE_BLK, None), lambda i: (i, 0))
    full = lambda s: pl.BlockSpec(s, lambda i: tuple(0 for _ in s))
    return pl.pallas_call(
        _node_kernel,
        grid=(nblk,),
        in_specs=[
            pl.BlockSpec((NODE_BLK, ND), lambda i: (i, 0)),
            pl.BlockSpec((NODE_BLK, G), lambda i: (i, 0)),
            full((ND, H)), full((1, H)), full((1, H)), full((1, H)),
            full((H, H)), full((1, H)), full((H, H)),
            full((ND, H)), full((G, H)), full((ND, H)), full((G, H)),
        ],
        out_specs=[
            pl.BlockSpec((NODE_BLK, H), lambda i: (i, 0)),
            pl.BlockSpec((NODE_BLK, H), lambda i: (i, 0)),
        ],
        out_shape=[
            jax.ShapeDtypeStruct((N, H), jnp.int32),
            jax.ShapeDtypeStruct((N, H), jnp.int32),
        ],
    )(x, ne, w1n, b1n, gn, bn, w2n, b2n, wm, wgxc, wgnc, wgxr, wgnr)


def _edge_kernel(ea_ref, gc_ref, gr_ref, w1x2_ref, b1e_ref, ge_ref, be_ref,
                 w2ep_ref, b2ep_ref, b1g_ref, gg_ref, bg_ref,
                 w2g_ref, b2g_ref, msg_ref):
    ea = ea_ref[...]
    wa = gc_ref[...]
    wb = gr_ref[...]
    g1 = (lax.bitcast_convert_type(lax.shift_left(wa, 16), jnp.float32)
          + lax.bitcast_convert_type(lax.shift_left(wb, 16), jnp.float32))
    g2 = (lax.bitcast_convert_type(wa & jnp.int32(-65536), jnp.float32)
          + lax.bitcast_convert_type(wb & jnp.int32(-65536), jnp.float32))
    both = jnp.dot(ea, w1x2_ref[...], preferred_element_type=jnp.float32)
    eh = both[:, :H] + b1e_ref[...]
    gh = both[:, H:] + b1g_ref[...] + g1
    gh = _ln_tc(gh, gg_ref[...], bg_ref[...])
    gh = jnp.maximum(gh, 0.0).astype(jnp.bfloat16)
    gate = jnp.dot(gh, w2g_ref[...], preferred_element_type=jnp.float32) + b2g_ref[...]
    eh = _ln_tc(eh, ge_ref[...], be_ref[...])
    eh = jnp.maximum(eh, 0.0).astype(jnp.bfloat16)
    me = jnp.dot(eh, w2ep_ref[...], preferred_element_type=jnp.float32) + b2ep_ref[...]
    msg_ref[...] = (me + g2) * jax.nn.sigmoid(gate)


def _edge_msgs(ea, g_c, g_r, w1x2, b1e, ge, be, w2ep, b2ep, b1g, gg, bg,
               w2g, b2g):
    ne = ea.shape[0]
    assert ne % EDGE_BLK == 0
    nblk = ne // EDGE_BLK
    full = lambda s: pl.BlockSpec(s, lambda i: tuple(0 for _ in s))
    return pl.pallas_call(
        _edge_kernel,
        grid=(nblk,),
        in_specs=[
            pl.BlockSpec((EDGE_BLK, ED), lambda i: (i, 0)),
            pl.BlockSpec((EDGE_BLK, H), lambda i: (i, 0)),
            pl.BlockSpec((EDGE_BLK, H), lambda i: (i, 0)),
            full((ED, 2 * H)), full((1, H)), full((1, H)), full((1, H)),
            full((H, H)), full((1, H)),
            full((1, H)), full((1, H)), full((1, H)),
            full((H, H)), full((1, H)),
        ],
        out_specs=pl.BlockSpec((EDGE_BLK, H), lambda i: (i, 0)),
        out_shape=jax.ShapeDtypeStruct((ne, H), jnp.float32),
    )(ea, g_c, g_r, w1x2, b1e, ge, be, w2ep, b2ep, b1g, gg, bg, w2g, b2g)


def _sc_gather(tc, tr, col, row, dep):
    ne = col.shape[0]
    epw = ne // SC_NW
    g_full = epw // GC
    g_tail = epw - g_full * GC
    assert ne % SC_NW == 0 and g_full % 3 == 0 and g_tail in (0, 16)

    def body(dep_hbm, tc_hbm, tr_hbm, col_hbm, row_hbm, gc_hbm, gr_hbm,
             icol, irow, buf_a0, buf_b0, buf_a1, buf_b1, buf_a2, buf_b2,
             sem_g0, sem_g1, sem_g2, sem_w0, sem_w1, sem_w2):
        del dep_hbm
        wid = lax.axis_index("s") * SC_NC + lax.axis_index("c")
        e0 = wid * epw
        pltpu.sync_copy(col_hbm.at[pl.ds(e0, epw)], icol)
        pltpu.sync_copy(row_hbm.at[pl.ds(e0, epw)], irow)

        slots = ((buf_a0, buf_b0, sem_g0, sem_w0),
                 (buf_a1, buf_b1, sem_g1, sem_w1),
                 (buf_a2, buf_b2, sem_g2, sem_w2))

        def issue(k, slot):
            ba, bb, sg, _ = slots[slot]
            pltpu.async_copy(tc_hbm.at[icol.at[pl.ds(k * GC, GC)]], ba, sg)
            pltpu.async_copy(tr_hbm.at[irow.at[pl.ds(k * GC, GC)]], bb, sg)

        issue(0, 0)
        issue(1, 1)
        issue(2, 2)

        def round_(t, carry):
            for b in (0, 1, 2):
                k = 3 * t + b
                ba, bb, sg, sw = slots[b]
                pltpu.make_async_copy(tc_hbm.at[icol.at[pl.ds(0, GC)]], ba, sg).wait()
                pltpu.make_async_copy(tr_hbm.at[irow.at[pl.ds(0, GC)]], bb, sg).wait()
                pltpu.async_copy(ba, gc_hbm.at[pl.ds(e0 + k * GC, GC)], sw)
                pltpu.async_copy(bb, gr_hbm.at[pl.ds(e0 + k * GC, GC)], sw)

                @pl.when(k < g_full - 3)
                def _():
                    pltpu.make_async_copy(ba, gc_hbm.at[pl.ds(0, GC)], sw).wait()
                    pltpu.make_async_copy(bb, gr_hbm.at[pl.ds(0, GC)], sw).wait()
                    issue(k + 3, b)

            return carry

        lax.fori_loop(0, g_full // 3, round_, 0)

        for b in (0, 1, 2):
            ba, bb, sg, sw = slots[b]
            pltpu.make_async_copy(ba, gc_hbm.at[pl.ds(0, GC)], sw).wait()
            pltpu.make_async_copy(bb, gr_hbm.at[pl.ds(0, GC)], sw).wait()

        if g_tail:
            ba, bb, sg, sw = slots[0]
            tb = g_full * GC
            pltpu.async_copy(tc_hbm.at[icol.at[pl.ds(tb, g_tail)]],
                             ba.at[pl.ds(0, g_tail)], sg)
            pltpu.async_copy(tr_hbm.at[irow.at[pl.ds(tb, g_tail)]],
                             bb.at[pl.ds(0, g_tail)], sg)
            pltpu.make_async_copy(tc_hbm.at[icol.at[pl.ds(0, g_tail)]],
                                  ba.at[pl.ds(0, g_tail)], sg).wait()
            pltpu.make_async_copy(tr_hbm.at[irow.at[pl.ds(0, g_tail)]],
                                  bb.at[pl.ds(0, g_tail)], sg).wait()
            pltpu.sync_copy(ba.at[pl.ds(0, g_tail)],
                            gc_hbm.at[pl.ds(e0 + tb, g_tail)])
            pltpu.sync_copy(bb.at[pl.ds(0, g_tail)],
                            gr_hbm.at[pl.ds(e0 + tb, g_tail)])

    mesh = plsc.VectorSubcoreMesh(core_axis_name="c", subcore_axis_name="s",
                                  num_cores=SC_NC, num_subcores=SC_NS)
    return pl.kernel(
        body,
        out_type=[jax.ShapeDtypeStruct((ne, H), jnp.int32),
                  jax.ShapeDtypeStruct((ne, H), jnp.int32)],
        mesh=mesh,
        scratch_types=[
            pltpu.VMEM((epw,), jnp.int32),
            pltpu.VMEM((epw,), jnp.int32),
            pltpu.VMEM((GC, H), jnp.int32),
            pltpu.VMEM((GC, H), jnp.int32),
            pltpu.VMEM((GC, H), jnp.int32),
            pltpu.VMEM((GC, H), jnp.int32),
            pltpu.VMEM((GC, H), jnp.int32),
            pltpu.VMEM((GC, H), jnp.int32),
            pltpu.SemaphoreType.DMA,
            pltpu.SemaphoreType.DMA,
            pltpu.SemaphoreType.DMA,
            pltpu.SemaphoreType.DMA,
            pltpu.SemaphoreType.DMA,
            pltpu.SemaphoreType.DMA,
        ],
    )(dep, tc, tr, col, row)


def _make_sc_scatter_body(ne):
  def _sc_scatter_body(dep_hbm, msg_hbm, row_hbm, out_hbm, idx0, idx1, mbuf0,
                       mbuf1, accum, sem_m0, sem_m1):
    del dep_hbm
    core = lax.axis_index("c")
    sid = lax.axis_index("s")

    def zero_row(r, c2):
        for j in range(H // 16):
            mbuf0[r, pl.ds(j * 16, 16)] = jnp.zeros((16,), jnp.float32)
        return c2

    lax.fori_loop(0, GC, zero_row, 0)
    for i in range(SC_ROWS // GC):
        pltpu.sync_copy(mbuf0, accum.at[pl.ds(sid * SC_ROWS + i * GC, GC)])
    rem = SC_ROWS - (SC_ROWS // GC) * GC
    if rem:
        pltpu.sync_copy(mbuf0.at[pl.ds(0, rem)],
                        accum.at[pl.ds(sid * SC_ROWS + (SC_ROWS // GC) * GC,
                                       rem)])

    @pl.when(sid == 0)
    def _():
        pltpu.sync_copy(mbuf0.at[pl.ds(0, SC_REM)],
                        accum.at[pl.ds(SC_ROWS * SC_NS, SC_REM)])

    plsc.subcore_barrier()

    base0 = core * (ne // 2)
    s_chunks = (ne // 2) // GC
    s_kmax = -(-s_chunks // SC_NS)
    slots = ((mbuf0, idx0, sem_m0), (mbuf1, idx1, sem_m1))

    def issue(k, slot):
        mb, ix, sm = slots[slot]
        cid = k * SC_NS + sid
        pltpu.async_copy(msg_hbm.at[pl.ds(base0 + cid * GC, GC)], mb, sm)
        pltpu.async_copy(row_hbm.at[pl.ds(base0 + cid * GC, GC)], ix, sm)

    issue(0, 0)
    issue(1, 1)

    def round_(t, carry):
        for b in (0, 1):
            k = 2 * t + b
            cid = k * SC_NS + sid
            mb, ix, sm = slots[b]

            @pl.when(cid < s_chunks)
            def _():
                pltpu.make_async_copy(msg_hbm.at[pl.ds(0, GC)], mb, sm).wait()
                pltpu.make_async_copy(row_hbm.at[pl.ds(0, GC)], ix, sm).wait()
                pltpu.sync_copy(mb, accum.at[ix], add=True)

            @pl.when((k + 2) * SC_NS + sid < s_chunks)
            def _():
                issue(k + 2, b)

        return carry

    lax.fori_loop(0, (s_kmax + 1) // 2, round_, 0)
    plsc.subcore_barrier()
    pltpu.sync_copy(accum.at[pl.ds(sid * SC_ROWS, SC_ROWS)],
                    out_hbm.at[core, pl.ds(sid * SC_ROWS, SC_ROWS)])

    @pl.when(sid == 0)
    def _():
        pltpu.sync_copy(accum.at[pl.ds(SC_ROWS * SC_NS, SC_REM)],
                        out_hbm.at[core, pl.ds(SC_ROWS * SC_NS, SC_REM)])

  return _sc_scatter_body


def _sc_scatter(msg, row, dep):
    ne = msg.shape[0]
    assert ne % (2 * GC) == 0
    mesh = plsc.VectorSubcoreMesh(core_axis_name="c", subcore_axis_name="s",
                                  num_cores=SC_NC, num_subcores=SC_NS)
    return pl.kernel(
        _make_sc_scatter_body(ne),
        out_type=jax.ShapeDtypeStruct((SC_NC, N, H), jnp.float32),
        mesh=mesh,
        scratch_types=[
            pltpu.VMEM((GC,), jnp.int32),
            pltpu.VMEM((GC,), jnp.int32),
            pltpu.VMEM((GC, H), jnp.float32),
            pltpu.VMEM((GC, H), jnp.float32),
            pltpu.VMEM_SHARED((N + 16, H), jnp.float32),
            pltpu.SemaphoreType.DMA,
            pltpu.SemaphoreType.DMA,
        ],
    )(dep, msg, row)


def _out_kernel(x_ref, *refs):
    (*a_refs, wc_ref, bc_ref, w1o_ref, b1o_ref, go_ref, bo_ref, w2o_ref,
     b2o_ref, lng_ref, lnb_ref, out_ref) = refs
    x = x_ref[...]
    aggr = a_refs[0][...]
    for ar in a_refs[1:]:
        aggr = aggr + ar[...]
    h = (jnp.dot(x, wc_ref[...], preferred_element_type=jnp.float32)
         + bc_ref[...] + aggr)
    h = jnp.dot(h, w1o_ref[...], preferred_element_type=jnp.float32) + b1o_ref[...]
    h = _ln_tc(h, go_ref[...], bo_ref[...])
    h = jnp.maximum(h, 0.0)
    h = jnp.dot(h, w2o_ref[...], preferred_element_type=jnp.float32) + b2o_ref[...]
    out_ref[...] = _ln_tc(h + x, lng_ref[...], lnb_ref[...])


def _out_mlp(x, aggrs, wc, bc, w1o, b1o, go, bo, w2o, b2o, lng, lnb):
    nblk = N // NODE_BLK
    full = lambda s: pl.BlockSpec(s, lambda i: tuple(0 for _ in s))
    return pl.pallas_call(
        _out_kernel,
        grid=(nblk,),
        in_specs=[
            pl.BlockSpec((NODE_BLK, ND), lambda i: (i, 0)),
        ] + [
            pl.BlockSpec((NODE_BLK, H), lambda i: (i, 0))
            for _ in aggrs
        ] + [
            full((ND, H)), full((1, H)),
            full((H, H)), full((1, H)), full((1, H)), full((1, H)),
            full((H, ND)), full((1, ND)), full((1, ND)), full((1, ND)),
        ],
        out_specs=pl.BlockSpec((NODE_BLK, ND), lambda i: (i, 0)),
        out_shape=jax.ShapeDtypeStruct((N, ND), jnp.float32),
    )(x, *aggrs, wc, bc, w1o, b1o, go, bo, w2o, b2o, lng, lnb)


def kernel(x, edge_index, edge_attr, node_extra, params):
    row = edge_index[0].astype(jnp.int32)
    col = edge_index[1].astype(jnp.int32)

    w1n, b1n, gn, bn, w2n, b2n = params['node_net']
    w1e, b1e, ge, be, w2e, b2e = params['edge_net']
    w1g, b1g, gg, bg, w2g, b2g = params['gate']
    wm, bm = params['msg']
    wc, bc = params['cent']
    lng, lnb = params['ln']

    r2 = lambda v: v.reshape(1, -1)
    wgea = w1g[0:ED]
    wgxc = w1g[ED:ED + ND]
    wgnc = w1g[ED + ND:ED + ND + G]
    wgxr = w1g[ED + ND + G:ED + 2 * ND + G]
    wgnr = w1g[ED + 2 * ND + G:]
    w2ep = w2e @ wm
    b2ep = b2e @ wm + bm

    tc, tr = _node_tables(x, node_extra, w1n, r2(b1n), r2(gn), r2(bn), w2n,
                          r2(b2n), wm, wgxc, wgnc, wgxr, wgnr)

    bf = lambda v: v.astype(jnp.bfloat16)
    w1x2 = jnp.concatenate([w1e, wgea], axis=1)
    pad = EP - E
    if pad:
        ea_p = jnp.concatenate([bf(edge_attr),
                                jnp.zeros((pad, ED), jnp.bfloat16)])
        spread = jnp.arange(pad, dtype=jnp.int32) % N
        col_p = jnp.concatenate([col, spread])
        row_g = jnp.concatenate([row, spread])
        row_s = jnp.concatenate([row, jnp.full((pad,), N, jnp.int32)])
    else:
        ea_p, col_p, row_g, row_s = bf(edge_attr), col, row, row

    partials = []
    sc_dep = x[:8, :8]
    msgs = []
    for s in range(N_SLICE):
        sl = slice(s * EPS, (s + 1) * EPS)
        g_c, g_r = _sc_gather(tc, tr, col_p[sl], row_g[sl], sc_dep)
        sc_dep = g_c
        msg = _edge_msgs(ea_p[sl], g_c, g_r, bf(w1x2), r2(b1e), r2(ge),
                         r2(be), bf(w2ep), r2(b2ep), r2(b1g), r2(gg), r2(bg),
                         bf(w2g), r2(b2g))
        msgs.append((msg, row_s[sl]))
    for msg, rw in msgs:
        p = _sc_scatter(msg, rw, sc_dep)
        sc_dep = p
        partials.extend([p[0], p[1]])

    w1o, b1o, go, bo, w2o, b2o = params['out_layer']
    return _out_mlp(x, partials, wc, r2(bc), w1o, r2(b1o),
                    r2(go), r2(bo), w2o, r2(b2o), r2(lng), r2(lnb))


# speedup vs baseline: 1.0839x; 1.0117x over previous
import functools

import jax
import jax.numpy as jnp
from jax import lax
from jax.experimental import pallas as pl
from jax.experimental.pallas import tpu as pltpu
from jax.experimental.pallas import tpu_sc as plsc

N = 10000
E = 320000
ND = 128
ED = 16
H = 128
G = 16

NODE_BLK = 2000
EDGE_BLK = 8000
EP = E
N_SLICE = 1
EPS = EP // N_SLICE

SC_NC = 2
SC_NS = 16
SC_NW = SC_NC * SC_NS
GC = 128
EPW = E // SC_NW
G_FULL = EPW // GC
G_TAIL = EPW - G_FULL * GC
SC_ROWS = 624
SC_REM = N - SC_ROWS * SC_NS
S_CHUNKS = (E // 2) // GC
S_KMAX = -(-S_CHUNKS // SC_NS)


def _ln_tc(h, g, b):
    m = jnp.mean(h, axis=-1, keepdims=True)
    v = jnp.mean((h - m) * (h - m), axis=-1, keepdims=True)
    return (h - m) * jax.lax.rsqrt(v + 1e-5) * g + b


def _node_kernel(x_ref, ne_ref, w1n_ref, b1n_ref, gn_ref, bn_ref, w2n_ref,
                 b2n_ref, wm_ref, wgxc_ref, wgnc_ref, wgxr_ref, wgnr_ref,
                 tc_ref, tr_ref):
    x = x_ref[...]
    ne = ne_ref[...]
    h = jnp.dot(x, w1n_ref[...], preferred_element_type=jnp.float32) + b1n_ref[...]
    h = _ln_tc(h, gn_ref[...], bn_ref[...])
    h = jnp.maximum(h, 0.0)
    h_node = jnp.dot(h, w2n_ref[...], preferred_element_type=jnp.float32) + b2n_ref[...]
    hnm = jnp.dot(h_node, wm_ref[...], preferred_element_type=jnp.float32)
    a = (jnp.dot(x, wgxc_ref[...], preferred_element_type=jnp.float32)
         + jnp.dot(ne, wgnc_ref[...], preferred_element_type=jnp.float32))
    b = (jnp.dot(x, wgxr_ref[...], preferred_element_type=jnp.float32)
         + jnp.dot(ne, wgnr_ref[...], preferred_element_type=jnp.float32))
    hi = lax.bitcast_convert_type(hnm, jnp.int32)
    ph = (hi + 0x8000) & jnp.int32(-65536)
    ai = lax.bitcast_convert_type(a, jnp.int32)
    bi = lax.bitcast_convert_type(b, jnp.int32)
    pa = lax.shift_right_logical(ai + 0x8000, 16)
    pb = lax.shift_right_logical(bi + 0x8000, 16)
    tc_ref[...] = ph | pa
    tr_ref[...] = ph | pb


def _node_tables(x, ne, w1n, b1n, gn, bn, w2n, b2n, wm, wgxc, wgnc, wgxr, wgnr):
    nblk = N // NODE_BLK
    row_spec = pl.BlockSpec((NODE_BLK, None), lambda i: (i, 0))
    full = lambda s: pl.BlockSpec(s, lambda i: tuple(0 for _ in s))
    return pl.pallas_call(
        _node_kernel,
        grid=(nblk,),
        in_specs=[
            pl.BlockSpec((NODE_BLK, ND), lambda i: (i, 0)),
            pl.BlockSpec((NODE_BLK, G), lambda i: (i, 0)),
            full((ND, H)), full((1, H)), full((1, H)), full((1, H)),
            full((H, H)), full((1, H)), full((H, H)),
            full((ND, H)), full((G, H)), full((ND, H)), full((G, H)),
        ],
        out_specs=[
            pl.BlockSpec((NODE_BLK, H), lambda i: (i, 0)),
            pl.BlockSpec((NODE_BLK, H), lambda i: (i, 0)),
        ],
        out_shape=[
            jax.ShapeDtypeStruct((N, H), jnp.int32),
            jax.ShapeDtypeStruct((N, H), jnp.int32),
        ],
    )(x, ne, w1n, b1n, gn, bn, w2n, b2n, wm, wgxc, wgnc, wgxr, wgnr)


def _edge_kernel(ea_ref, gc_ref, gr_ref, w1x2_ref, b1e_ref, ge_ref, be_ref,
                 w2ep_ref, b2ep_ref, b1g_ref, gg_ref, bg_ref,
                 w2g_ref, b2g_ref, msg_ref):
    ea = ea_ref[...]
    wa = gc_ref[...]
    wb = gr_ref[...]
    g1 = (lax.bitcast_convert_type(lax.shift_left(wa, 16), jnp.float32)
          + lax.bitcast_convert_type(lax.shift_left(wb, 16), jnp.float32))
    g2 = (lax.bitcast_convert_type(wa & jnp.int32(-65536), jnp.float32)
          + lax.bitcast_convert_type(wb & jnp.int32(-65536), jnp.float32))
    both = jnp.dot(ea, w1x2_ref[...], preferred_element_type=jnp.float32)
    eh = both[:, :H] + b1e_ref[...]
    gh = both[:, H:] + b1g_ref[...] + g1
    gh = _ln_tc(gh, gg_ref[...], bg_ref[...])
    gh = jnp.maximum(gh, 0.0).astype(jnp.bfloat16)
    gate = jnp.dot(gh, w2g_ref[...], preferred_element_type=jnp.float32) + b2g_ref[...]
    eh = _ln_tc(eh, ge_ref[...], be_ref[...])
    eh = jnp.maximum(eh, 0.0).astype(jnp.bfloat16)
    me = jnp.dot(eh, w2ep_ref[...], preferred_element_type=jnp.float32) + b2ep_ref[...]
    msg_ref[...] = (me + g2) * jax.nn.sigmoid(gate)


def _edge_msgs(ea, g_c, g_r, w1x2, b1e, ge, be, w2ep, b2ep, b1g, gg, bg,
               w2g, b2g):
    ne = ea.shape[0]
    assert ne % EDGE_BLK == 0
    nblk = ne // EDGE_BLK
    full = lambda s: pl.BlockSpec(s, lambda i: tuple(0 for _ in s))
    return pl.pallas_call(
        _edge_kernel,
        grid=(nblk,),
        in_specs=[
            pl.BlockSpec((EDGE_BLK, ED), lambda i: (i, 0)),
            pl.BlockSpec((EDGE_BLK, H), lambda i: (i, 0)),
            pl.BlockSpec((EDGE_BLK, H), lambda i: (i, 0)),
            full((ED, 2 * H)), full((1, H)), full((1, H)), full((1, H)),
            full((H, H)), full((1, H)),
            full((1, H)), full((1, H)), full((1, H)),
            full((H, H)), full((1, H)),
        ],
        out_specs=pl.BlockSpec((EDGE_BLK, H), lambda i: (i, 0)),
        out_shape=jax.ShapeDtypeStruct((ne, H), jnp.float32),
    )(ea, g_c, g_r, w1x2, b1e, ge, be, w2ep, b2ep, b1g, gg, bg, w2g, b2g)


def _sc_gather(tc, tr, col, row, dep):
    ne = col.shape[0]
    epw = ne // SC_NW
    g_full = epw // GC
    g_tail = epw - g_full * GC
    assert ne % SC_NW == 0 and g_full % 3 == 0 and g_tail in (0, 16)

    def body(dep_hbm, tc_hbm, tr_hbm, col_hbm, row_hbm, gc_hbm, gr_hbm,
             icol, irow, buf_a0, buf_b0, buf_a1, buf_b1, buf_a2, buf_b2,
             sem_g0, sem_g1, sem_g2, sem_w0, sem_w1, sem_w2):
        del dep_hbm
        wid = lax.axis_index("s") * SC_NC + lax.axis_index("c")
        e0 = wid * epw
        pltpu.sync_copy(col_hbm.at[pl.ds(e0, epw)], icol)
        pltpu.sync_copy(row_hbm.at[pl.ds(e0, epw)], irow)

        slots = ((buf_a0, buf_b0, sem_g0, sem_w0),
                 (buf_a1, buf_b1, sem_g1, sem_w1),
                 (buf_a2, buf_b2, sem_g2, sem_w2))

        def issue(k, slot):
            ba, bb, sg, _ = slots[slot]
            pltpu.async_copy(tc_hbm.at[icol.at[pl.ds(k * GC, GC)]], ba, sg)
            pltpu.async_copy(tr_hbm.at[irow.at[pl.ds(k * GC, GC)]], bb, sg)

        issue(0, 0)
        issue(1, 1)
        issue(2, 2)

        def round_(t, carry):
            for b in (0, 1, 2):
                k = 3 * t + b
                ba, bb, sg, sw = slots[b]
                pltpu.make_async_copy(tc_hbm.at[icol.at[pl.ds(0, GC)]], ba, sg).wait()
                pltpu.make_async_copy(tr_hbm.at[irow.at[pl.ds(0, GC)]], bb, sg).wait()
                pltpu.async_copy(ba, gc_hbm.at[pl.ds(e0 + k * GC, GC)], sw)
                pltpu.async_copy(bb, gr_hbm.at[pl.ds(e0 + k * GC, GC)], sw)

                @pl.when(k < g_full - 3)
                def _():
                    pltpu.make_async_copy(ba, gc_hbm.at[pl.ds(0, GC)], sw).wait()
                    pltpu.make_async_copy(bb, gr_hbm.at[pl.ds(0, GC)], sw).wait()
                    issue(k + 3, b)

            return carry

        lax.fori_loop(0, g_full // 3, round_, 0)

        for b in (0, 1, 2):
            ba, bb, sg, sw = slots[b]
            pltpu.make_async_copy(ba, gc_hbm.at[pl.ds(0, GC)], sw).wait()
            pltpu.make_async_copy(bb, gr_hbm.at[pl.ds(0, GC)], sw).wait()

        if g_tail:
            ba, bb, sg, sw = slots[0]
            tb = g_full * GC
            pltpu.async_copy(tc_hbm.at[icol.at[pl.ds(tb, g_tail)]],
                             ba.at[pl.ds(0, g_tail)], sg)
            pltpu.async_copy(tr_hbm.at[irow.at[pl.ds(tb, g_tail)]],
                             bb.at[pl.ds(0, g_tail)], sg)
            pltpu.make_async_copy(tc_hbm.at[icol.at[pl.ds(0, g_tail)]],
                                  ba.at[pl.ds(0, g_tail)], sg).wait()
            pltpu.make_async_copy(tr_hbm.at[irow.at[pl.ds(0, g_tail)]],
                                  bb.at[pl.ds(0, g_tail)], sg).wait()
            pltpu.sync_copy(ba.at[pl.ds(0, g_tail)],
                            gc_hbm.at[pl.ds(e0 + tb, g_tail)])
            pltpu.sync_copy(bb.at[pl.ds(0, g_tail)],
                            gr_hbm.at[pl.ds(e0 + tb, g_tail)])

    mesh = plsc.VectorSubcoreMesh(core_axis_name="c", subcore_axis_name="s",
                                  num_cores=SC_NC, num_subcores=SC_NS)
    return pl.kernel(
        body,
        out_type=[jax.ShapeDtypeStruct((ne, H), jnp.int32),
                  jax.ShapeDtypeStruct((ne, H), jnp.int32)],
        mesh=mesh,
        scratch_types=[
            pltpu.VMEM((epw,), jnp.int32),
            pltpu.VMEM((epw,), jnp.int32),
            pltpu.VMEM((GC, H), jnp.int32),
            pltpu.VMEM((GC, H), jnp.int32),
            pltpu.VMEM((GC, H), jnp.int32),
            pltpu.VMEM((GC, H), jnp.int32),
            pltpu.VMEM((GC, H), jnp.int32),
            pltpu.VMEM((GC, H), jnp.int32),
            pltpu.SemaphoreType.DMA,
            pltpu.SemaphoreType.DMA,
            pltpu.SemaphoreType.DMA,
            pltpu.SemaphoreType.DMA,
            pltpu.SemaphoreType.DMA,
            pltpu.SemaphoreType.DMA,
        ],
    )(dep, tc, tr, col, row)


def _make_sc_scatter_body(ne):
  def _sc_scatter_body(dep_hbm, msg_hbm, row_hbm, out_hbm, idx0, idx1, mbuf0,
                       mbuf1, accum, sem_m0, sem_m1):
    del dep_hbm
    core = lax.axis_index("c")
    sid = lax.axis_index("s")

    def zero_row(r, c2):
        for j in range(H // 16):
            mbuf0[r, pl.ds(j * 16, 16)] = jnp.zeros((16,), jnp.float32)
        return c2

    lax.fori_loop(0, GC, zero_row, 0)
    for i in range(SC_ROWS // GC):
        pltpu.sync_copy(mbuf0, accum.at[pl.ds(sid * SC_ROWS + i * GC, GC)])
    rem = SC_ROWS - (SC_ROWS // GC) * GC
    if rem:
        pltpu.sync_copy(mbuf0.at[pl.ds(0, rem)],
                        accum.at[pl.ds(sid * SC_ROWS + (SC_ROWS // GC) * GC,
                                       rem)])

    @pl.when(sid == 0)
    def _():
        pltpu.sync_copy(mbuf0.at[pl.ds(0, SC_REM)],
                        accum.at[pl.ds(SC_ROWS * SC_NS, SC_REM)])

    plsc.subcore_barrier()

    base0 = core * (ne // 2)
    s_chunks = (ne // 2) // GC
    s_kmax = -(-s_chunks // SC_NS)
    slots = ((mbuf0, idx0, sem_m0), (mbuf1, idx1, sem_m1))

    def issue(k, slot):
        mb, ix, sm = slots[slot]
        cid = k * SC_NS + sid
        pltpu.async_copy(msg_hbm.at[pl.ds(base0 + cid * GC, GC)], mb, sm)
        pltpu.async_copy(row_hbm.at[pl.ds(base0 + cid * GC, GC)], ix, sm)

    issue(0, 0)
    issue(1, 1)

    def round_(t, carry):
        for b in (0, 1):
            k = 2 * t + b
            cid = k * SC_NS + sid
            mb, ix, sm = slots[b]

            @pl.when(cid < s_chunks)
            def _():
                pltpu.make_async_copy(msg_hbm.at[pl.ds(0, GC)], mb, sm).wait()
                pltpu.make_async_copy(row_hbm.at[pl.ds(0, GC)], ix, sm).wait()
                pltpu.sync_copy(mb, accum.at[ix], add=True)

            @pl.when((k + 2) * SC_NS + sid < s_chunks)
            def _():
                issue(k + 2, b)

        return carry

    lax.fori_loop(0, (s_kmax + 1) // 2, round_, 0)
    plsc.subcore_barrier()
    pltpu.sync_copy(accum.at[pl.ds(sid * SC_ROWS, SC_ROWS)],
                    out_hbm.at[core, pl.ds(sid * SC_ROWS, SC_ROWS)])

    @pl.when(sid == 0)
    def _():
        pltpu.sync_copy(accum.at[pl.ds(SC_ROWS * SC_NS, SC_REM)],
                        out_hbm.at[core, pl.ds(SC_ROWS * SC_NS, SC_REM)])

  return _sc_scatter_body


def _sc_scatter(msg, row, dep):
    ne = msg.shape[0]
    assert ne % (2 * GC) == 0
    mesh = plsc.VectorSubcoreMesh(core_axis_name="c", subcore_axis_name="s",
                                  num_cores=SC_NC, num_subcores=SC_NS)
    return pl.kernel(
        _make_sc_scatter_body(ne),
        out_type=jax.ShapeDtypeStruct((SC_NC, N, H), jnp.float32),
        mesh=mesh,
        scratch_types=[
            pltpu.VMEM((GC,), jnp.int32),
            pltpu.VMEM((GC,), jnp.int32),
            pltpu.VMEM((GC, H), jnp.float32),
            pltpu.VMEM((GC, H), jnp.float32),
            pltpu.VMEM_SHARED((N + 16, H), jnp.float32),
            pltpu.SemaphoreType.DMA,
            pltpu.SemaphoreType.DMA,
        ],
    )(dep, msg, row)


def _out_kernel(x_ref, *refs):
    (*a_refs, wc_ref, bc_ref, w1o_ref, b1o_ref, go_ref, bo_ref, w2o_ref,
     b2o_ref, lng_ref, lnb_ref, out_ref) = refs
    x = x_ref[...]
    aggr = a_refs[0][...]
    for ar in a_refs[1:]:
        aggr = aggr + ar[...]
    h = (jnp.dot(x, wc_ref[...], preferred_element_type=jnp.float32)
         + bc_ref[...] + aggr)
    h = jnp.dot(h, w1o_ref[...], preferred_element_type=jnp.float32) + b1o_ref[...]
    h = _ln_tc(h, go_ref[...], bo_ref[...])
    h = jnp.maximum(h, 0.0)
    h = jnp.dot(h, w2o_ref[...], preferred_element_type=jnp.float32) + b2o_ref[...]
    out_ref[...] = _ln_tc(h + x, lng_ref[...], lnb_ref[...])


def _out_mlp(x, aggrs, wc, bc, w1o, b1o, go, bo, w2o, b2o, lng, lnb):
    nblk = N // NODE_BLK
    full = lambda s: pl.BlockSpec(s, lambda i: tuple(0 for _ in s))
    return pl.pallas_call(
        _out_kernel,
        grid=(nblk,),
        in_specs=[
            pl.BlockSpec((NODE_BLK, ND), lambda i: (i, 0)),
        ] + [
            pl.BlockSpec((NODE_BLK, H), lambda i: (i, 0))
            for _ in aggrs
        ] + [
            full((ND, H)), full((1, H)),
            full((H, H)), full((1, H)), full((1, H)), full((1, H)),
            full((H, ND)), full((1, ND)), full((1, ND)), full((1, ND)),
        ],
        out_specs=pl.BlockSpec((NODE_BLK, ND), lambda i: (i, 0)),
        out_shape=jax.ShapeDtypeStruct((N, ND), jnp.float32),
    )(x, *aggrs, wc, bc, w1o, b1o, go, bo, w2o, b2o, lng, lnb)


def kernel(x, edge_index, edge_attr, node_extra, params):
    row = edge_index[0].astype(jnp.int32)
    col = edge_index[1].astype(jnp.int32)

    w1n, b1n, gn, bn, w2n, b2n = params['node_net']
    w1e, b1e, ge, be, w2e, b2e = params['edge_net']
    w1g, b1g, gg, bg, w2g, b2g = params['gate']
    wm, bm = params['msg']
    wc, bc = params['cent']
    lng, lnb = params['ln']

    r2 = lambda v: v.reshape(1, -1)
    wgea = w1g[0:ED]
    wgxc = w1g[ED:ED + ND]
    wgnc = w1g[ED + ND:ED + ND + G]
    wgxr = w1g[ED + ND + G:ED + 2 * ND + G]
    wgnr = w1g[ED + 2 * ND + G:]
    w2ep = w2e @ wm
    b2ep = b2e @ wm + bm

    tc, tr = _node_tables(x, node_extra, w1n, r2(b1n), r2(gn), r2(bn), w2n,
                          r2(b2n), wm, wgxc, wgnc, wgxr, wgnr)

    bf = lambda v: v.astype(jnp.bfloat16)
    w1x2 = jnp.concatenate([w1e, wgea], axis=1)
    pad = EP - E
    if pad:
        ea_p = jnp.concatenate([bf(edge_attr),
                                jnp.zeros((pad, ED), jnp.bfloat16)])
        spread = jnp.arange(pad, dtype=jnp.int32) % N
        col_p = jnp.concatenate([col, spread])
        row_g = jnp.concatenate([row, spread])
        row_s = jnp.concatenate([row, jnp.full((pad,), N, jnp.int32)])
    else:
        ea_p, col_p, row_g, row_s = bf(edge_attr), col, row, row

    partials = []
    sc_dep = x[:8, :8]
    msgs = []
    for s in range(N_SLICE):
        sl = slice(s * EPS, (s + 1) * EPS)
        g_c, g_r = _sc_gather(tc, tr, col_p[sl], row_g[sl], sc_dep)
        sc_dep = g_c
        msg = _edge_msgs(ea_p[sl], g_c, g_r, bf(w1x2), r2(b1e), r2(ge),
                         r2(be), bf(w2ep), r2(b2ep), r2(b1g), r2(gg), r2(bg),
                         bf(w2g), r2(b2g))
        msgs.append((msg, row_s[sl]))
    for msg, rw in msgs:
        p = _sc_scatter(msg, rw, sc_dep)
        sc_dep = p
        partials.extend([p[0], p[1]])

    w1o, b1o, go, bo, w2o, b2o = params['out_layer']
    return _out_mlp(x, partials, wc, r2(bc), w1o, r2(b1o),
                    r2(go), r2(bo), w2o, r2(b2o), r2(lng), r2(lnb))
